# Initial kernel scaffold; baseline (speedup 1.0000x reference)
import jax
import jax.numpy as jnp
from jax.experimental import pallas as pl


def kernel(token_ids, basin_coords, W, rms_weight):
    raise NotImplementedError("write your pallas kernel here")


# speedup vs baseline: 1.3798x; 1.3798x over previous
import functools

import jax
import jax.numpy as jnp
from jax import lax
from jax.experimental import pallas as pl
from jax.experimental.pallas import tpu as pltpu
from jax.experimental.pallas import tpu_sc as plsc

_IDX_CHUNK = 128


def _sc_gather(table, ids3):
    num_workers, chunks_per_w, chunk = ids3.shape
    depth = table.shape[1]
    total = num_workers * chunks_per_w * chunk

    info = plsc.get_sparse_core_info()
    num_cores = info.num_cores

    mesh = plsc.VectorSubcoreMesh(core_axis_name="c", subcore_axis_name="s")

    @functools.partial(
        pl.kernel,
        mesh=mesh,
        out_type=jax.ShapeDtypeStruct((total, depth), jnp.float32),
        scratch_types=[
            pltpu.VMEM((chunks_per_w, chunk), jnp.int32),
            pltpu.VMEM((chunk, depth), jnp.float32),
            pltpu.SemaphoreType.DMA,
        ],
        compiler_params=pltpu.CompilerParams(use_tc_tiling_on_sc=False),
    )
    def gather_kernel(table_hbm, idx_hbm, out_hbm, idx_v, rows_v, sem):
        wid = lax.axis_index("s") * num_cores + lax.axis_index("c")
        first_chunk = wid * chunks_per_w
        pltpu.sync_copy(idx_hbm.at[wid], idx_v)

        def body(j, carry):
            pltpu.async_copy(table_hbm.at[idx_v.at[j]], rows_v, sem).wait()
            off = pl.multiple_of((first_chunk + j) * chunk, chunk)
            pltpu.sync_copy(rows_v, out_hbm.at[pl.ds(off, chunk)])
            return carry

        lax.fori_loop(0, chunks_per_w, body, 0)

    return gather_kernel(table, ids3)


def _tc_project_norm(gathered, W, rms_weight, block_tokens):
    total, depth = gathered.shape
    d_model = W.shape[0]
    grid = total // block_tokens

    def body(g_ref, w_ref, rw_ref, o_ref):
        y = lax.dot_general(
            g_ref[...], w_ref[...], (((1,), (1,)), ((), ())),
            preferred_element_type=jnp.float32,
        )
        ms = jnp.mean(y * y, axis=-1, keepdims=True)
        o_ref[...] = y * lax.rsqrt(ms + 1e-8) * rw_ref[...]

    return pl.pallas_call(
        body,
        grid=(grid,),
        in_specs=[
            pl.BlockSpec((block_tokens, depth), lambda i: (i, 0)),
            pl.BlockSpec((d_model, depth), lambda i: (0, 0)),
            pl.BlockSpec((1, d_model), lambda i: (0, 0)),
        ],
        out_specs=pl.BlockSpec((block_tokens, d_model), lambda i: (i, 0)),
        out_shape=jax.ShapeDtypeStruct((total, d_model), jnp.float32),
        compiler_params=pltpu.CompilerParams(
            dimension_semantics=("arbitrary",),
        ),
    )(gathered, W, rms_weight)


def kernel(token_ids, basin_coords, W, rms_weight):
    batch, seq = token_ids.shape
    d_model = W.shape[0]
    info = plsc.get_sparse_core_info()
    num_workers = info.num_cores * info.num_subcores
    ids = token_ids.reshape(-1).astype(jnp.int32)
    ids3 = ids.reshape(num_workers, -1, _IDX_CHUNK)
    gathered = _sc_gather(basin_coords, ids3)
    out = _tc_project_norm(gathered, W, rms_weight.reshape(1, d_model), 2048)
    return out.reshape(batch, seq, d_model)
